# baseline (speedup 1.0000x reference)
import jax
import jax.numpy as jnp
from jax import lax
from jax.experimental import pallas as pl
from jax.experimental.pallas import tpu as pltpu
from jax.experimental.pallas import tpu_sc as plsc

_N = 1000000
_D = 64
_NV = _N // 2
_NW = 32
_VROWS = 15624
_VTAIL = _NV - _NW * _VROWS
_CCH = 248
_NCH = _VROWS // _CCH


def _body(sv_hbm, out_hbm, cbuf, sem_in, sem_out):
    c = lax.axis_index("c")
    s = lax.axis_index("s")
    wid = s * 2 + c
    base = wid * _VROWS

    def in_copy(i, slot):
        return pltpu.make_async_copy(
            sv_hbm.at[pl.ds(base + i * _CCH, _CCH)],
            cbuf.at[slot], sem_in.at[slot])

    def out_copy(i, slot):
        return pltpu.make_async_copy(
            cbuf.at[slot],
            out_hbm.at[pl.ds(base + i * _CCH, _CCH)], sem_out.at[slot])

    in_copy(0, 0).start()

    def copy_step(i, _):
        slot = lax.rem(i, 2)
        nxt = lax.rem(i + 1, 2)

        @pl.when(i + 1 < _NCH)
        def _():
            @pl.when(i >= 1)
            def _():
                out_copy(i - 1, nxt).wait()
            in_copy(i + 1, nxt).start()

        in_copy(i, slot).wait()
        out_copy(i, slot).start()
        return 0

    lax.fori_loop(0, _NCH, copy_step, 0)
    out_copy(_NCH - 2, _NCH % 2).wait()
    out_copy(_NCH - 1, 1 - _NCH % 2).wait()

    @pl.when(wid == _NW - 1)
    def _():
        pltpu.sync_copy(sv_hbm.at[pl.ds(_NW * _VROWS, _VTAIL)],
                        out_hbm.at[pl.ds(_NW * _VROWS, _VTAIL)])


def kernel(states, idx, updated):
    sv = jnp.reshape(states, (_NV, 128))
    mesh = plsc.VectorSubcoreMesh(core_axis_name="c", subcore_axis_name="s")
    copied = pl.kernel(
        _body,
        out_type=jax.ShapeDtypeStruct((_NV, 128), jnp.float32),
        mesh=mesh,
        scratch_types=[
            pltpu.VMEM((2, _CCH, 128), jnp.float32),
            pltpu.SemaphoreType.DMA((2,)),
            pltpu.SemaphoreType.DMA((2,)),
        ],
    )(sv)
    return jnp.reshape(copied, (_N, _D)).at[idx].set(updated)


# speedup vs baseline: 1.1828x; 1.1828x over previous
import jax
import jax.numpy as jnp
from jax import lax
from jax.experimental import pallas as pl
from jax.experimental.pallas import tpu as pltpu
from jax.experimental.pallas import tpu_sc as plsc

_N = 1000000
_D = 64
_B = 16384
_NW = 32
_ROWS = 31248
_TAIL = _N - _NW * _ROWS
_RANGE_LAST = _ROWS + _TAIL
_CAP = 1024
_CCH = 248
_NCH = _ROWS // _CCH
_SCAP = 96
_ICH = 2048


def _psum16(mi):
    return plsc.cumsum(mi)


def _body(states_hbm, idx_hbm, uv_hbm, out_hbm,
          idx_v, tbuf, pbuf, tfin, pfin, posmap, cbuf, stage, subt, subp,
          sem_in, sem_out, sem_g):
    c = lax.axis_index("c")
    s = lax.axis_index("s")
    wid = s * 2 + c
    base = wid * _ROWS
    myrange = jnp.where(wid == _NW - 1, _RANGE_LAST, _ROWS)
    lanes = lax.broadcasted_iota(jnp.int32, (16,), 0)

    def scan_outer(ci, cnt):
        pltpu.sync_copy(idx_hbm.at[pl.ds(ci * _ICH, _ICH)], idx_v)

        def scan_step(k, cnt):
            t = idx_v[pl.ds(k * 16, 16)] - base
            m = (t >= 0) & (t < myrange)
            off = _psum16(jnp.where(m, 1, 0).astype(jnp.int32))
            dest = jnp.maximum(cnt + off - 1, 0)
            plsc.store_scatter(tbuf, [dest], t, mask=m)
            plsc.store_scatter(pbuf, [dest], ci * _ICH + k * 16 + lanes, mask=m)
            return jnp.minimum(cnt + off[15], _CAP - 16)

        return lax.fori_loop(0, _ICH // 16, scan_step, cnt)

    cnt = lax.fori_loop(0, _B // _ICH, scan_outer, jnp.int32(0))
    nch = (cnt + 15) // 16

    def dedup_step(j, _):
        valid = (j * 16 + lanes) < cnt
        tt = jnp.where(valid, tbuf[pl.ds(j * 16, 16)], 0)
        pp = jnp.where(valid, pbuf[pl.ds(j * 16, 16)], -1)

        def cond(lost):
            return jnp.any(lost)

        def body(lost):
            plsc.store_scatter(posmap, [tt], pp, mask=lost)
            g = plsc.load_gather(posmap, [tt], mask=valid)
            return valid & (g < pp)

        lax.while_loop(cond, body, valid)
        return 0

    lax.fori_loop(0, nch, dedup_step, 0)

    def win_step(j, cnt2):
        valid = (j * 16 + lanes) < cnt
        tt = jnp.where(valid, tbuf[pl.ds(j * 16, 16)], 0)
        pp = jnp.where(valid, pbuf[pl.ds(j * 16, 16)], -1)
        g = plsc.load_gather(posmap, [tt], mask=valid)
        keep = valid & (g == pp)
        off = _psum16(jnp.where(keep, 1, 0).astype(jnp.int32))
        dest = jnp.maximum(cnt2 + off - 1, 0)
        plsc.store_scatter(tfin, [dest], tt, mask=keep)
        plsc.store_scatter(pfin, [dest], pp, mask=keep)
        return jnp.minimum(cnt2 + off[15], _CAP - 16)

    cnt2 = lax.fori_loop(0, nch, win_step, jnp.int32(0))
    nj2 = (cnt2 + 15) // 16

    def prep(lo, hi, slot):
        def pstep(j, ns):
            valid = (j * 16 + lanes) < cnt2
            tt = jnp.where(valid, tfin[pl.ds(j * 16, 16)], -1)
            m = valid & (tt >= lo) & (tt < hi)
            off = _psum16(jnp.where(m, 1, 0).astype(jnp.int32))
            dest = jnp.minimum(jnp.maximum(ns + off - 1, 0), _SCAP + 15)
            srow = jnp.full((16,), slot, jnp.int32)
            plsc.store_scatter(subt, [srow, dest], tt - lo, mask=m)
            pp = jnp.where(valid, pfin[pl.ds(j * 16, 16)], 0)
            plsc.store_scatter(subp, [srow, dest], pp, mask=m)
            return jnp.minimum(ns + off[15], _SCAP)

        nsub = lax.fori_loop(0, nj2, pstep, jnp.int32(0))
        ngath = (nsub + 15) // 16

        def gstep(v, _):
            pv = jnp.clip(subp[slot, pl.ds(v * 16, 16)] // 2, 0, _B // 2 - 1)
            pltpu.async_copy(uv_hbm.at[pv], stage.at[slot].at[pl.ds(v * 16, 16)],
                             sem_g.at[slot])
            return 0

        lax.fori_loop(0, ngath, gstep, 0)
        return nsub, ngath

    def drain_gathers(ngath, slot):
        def dstep(v, _):
            pltpu.make_async_copy(uv_hbm.at[pl.ds(0, 16)],
                                  stage.at[slot].at[pl.ds(0, 16)],
                                  sem_g.at[slot]).wait()
            return 0

        lax.fori_loop(0, ngath, dstep, 0)

    def merge(nsub, slot, cslot):
        nv = (nsub + 15) // 16
        cb = cbuf.at[cslot]
        st = stage.at[slot]

        def mstep(v, _):
            valid = (v * 16 + lanes) < nsub
            rloc = jnp.where(valid, subt[slot, pl.ds(v * 16, 16)], 0)
            pp = jnp.where(valid, subp[slot, pl.ds(v * 16, 16)], 0)
            srow = v * 16 + lanes
            halfoff = (pp & 1) * _D

            def cstep(j, _):
                vals = plsc.load_gather(st, [srow, halfoff + j], mask=valid)
                plsc.store_scatter(cb, [rloc, jnp.full((16,), j, jnp.int32)],
                                   vals, mask=valid)
                return 0

            lax.fori_loop(0, _D, cstep, 0)
            return 0

        lax.fori_loop(0, nv, mstep, 0)

    def in_copy(i, slot):
        return pltpu.make_async_copy(
            states_hbm.at[pl.ds(base + i * _CCH, _CCH)],
            cbuf.at[slot], sem_in.at[slot])

    def out_copy(i, slot):
        return pltpu.make_async_copy(
            cbuf.at[slot],
            out_hbm.at[pl.ds(base + i * _CCH, _CCH)], sem_out.at[slot])

    in_copy(0, 0).start()
    ns0, ng0 = prep(0, _CCH, 0)

    def copy_step(i, carry):
        nsub_cur, ngath_cur = carry
        slot = lax.rem(i, 2)
        nxt = lax.rem(i + 1, 2)

        nsub_nxt = jnp.int32(0)
        ngath_nxt = jnp.int32(0)

        @pl.when(i + 1 < _NCH)
        def _():
            @pl.when(i >= 1)
            def _():
                out_copy(i - 1, nxt).wait()
            in_copy(i + 1, nxt).start()

        nsub_nxt, ngath_nxt = lax.cond(
            i + 1 < _NCH,
            lambda: prep((i + 1) * _CCH, (i + 2) * _CCH, nxt),
            lambda: (jnp.int32(0), jnp.int32(0)))

        in_copy(i, slot).wait()
        drain_gathers(ngath_cur, slot)
        merge(nsub_cur, slot, slot)
        out_copy(i, slot).start()
        return nsub_nxt, ngath_nxt

    lax.fori_loop(0, _NCH, copy_step, (ns0, ng0))
    out_copy(_NCH - 2, _NCH % 2).wait()
    out_copy(_NCH - 1, 1 - _NCH % 2).wait()

    @pl.when(wid == _NW - 1)
    def _():
        nst, ngt = prep(_ROWS, _RANGE_LAST, 0)
        pltpu.sync_copy(states_hbm.at[pl.ds(_NW * _ROWS, _TAIL)],
                        cbuf.at[0].at[pl.ds(0, _TAIL)])
        drain_gathers(ngt, 0)
        merge(nst, 0, 0)
        pltpu.sync_copy(cbuf.at[0].at[pl.ds(0, _TAIL)],
                        out_hbm.at[pl.ds(_NW * _ROWS, _TAIL)])


def kernel(states, idx, updated):
    uv = jnp.reshape(updated, (_B // 2, 2 * _D))
    mesh = plsc.VectorSubcoreMesh(core_axis_name="c", subcore_axis_name="s")
    return pl.kernel(
        _body,
        out_type=jax.ShapeDtypeStruct((_N, _D), jnp.float32),
        mesh=mesh,
        compiler_params=pltpu.CompilerParams(needs_layout_passes=False),
        scratch_types=[
            pltpu.VMEM((_ICH,), jnp.int32),
            pltpu.VMEM((_CAP,), jnp.int32),
            pltpu.VMEM((_CAP,), jnp.int32),
            pltpu.VMEM((_CAP,), jnp.int32),
            pltpu.VMEM((_CAP,), jnp.int32),
            pltpu.VMEM((_RANGE_LAST,), jnp.int32),
            pltpu.VMEM((2, _CCH, _D), jnp.float32),
            pltpu.VMEM((2, _SCAP + 16, 2 * _D), jnp.float32),
            pltpu.VMEM((2, _SCAP + 32), jnp.int32),
            pltpu.VMEM((2, _SCAP + 32), jnp.int32),
            pltpu.SemaphoreType.DMA((2,)),
            pltpu.SemaphoreType.DMA((2,)),
            pltpu.SemaphoreType.DMA((2,)),
        ],
    )(states, idx, uv)


# speedup vs baseline: 2.7592x; 2.3327x over previous
import jax
import jax.numpy as jnp
from jax import lax
from jax.experimental import pallas as pl
from jax.experimental.pallas import tpu as pltpu
from jax.experimental.pallas import tpu_sc as plsc

_N = 1000000
_D = 64
_B = 16384
_NW = 32
_ROWS = 31248
_TAIL = _N - _NW * _ROWS
_RANGE_LAST = _ROWS + _TAIL
_CAP = 1024
_CCH = 248
_NCH = _ROWS // _CCH
_NBIN = _NCH + 1
_SCAP = 96
_ICH = 2048


def _psum16(mi):
    return plsc.cumsum(mi)


def _body(states_hbm, idx_hbm, uv_hbm, out_hbm,
          idx_v, tbuf, pbuf, tfin, pfin, posmap, cbuf, stage,
          bincnt, binoff, ctr, claim,
          sem_in, sem_out, sem_g):
    c = lax.axis_index("c")
    s = lax.axis_index("s")
    wid = s * 2 + c
    base = wid * _ROWS
    myrange = jnp.where(wid == _NW - 1, _RANGE_LAST, _ROWS)
    lanes = lax.broadcasted_iota(jnp.int32, (16,), 0)

    def scan_outer(ci, cnt):
        pltpu.sync_copy(idx_hbm.at[pl.ds(ci * _ICH, _ICH)], idx_v)

        def scan_step(k, cnt):
            t = idx_v[pl.ds(k * 16, 16)] - base
            m = (t >= 0) & (t < myrange)

            def compact():
                off = _psum16(jnp.where(m, 1, 0).astype(jnp.int32))
                dest = jnp.maximum(cnt + off - 1, 0)
                plsc.store_scatter(tbuf, [dest], t, mask=m)
                plsc.store_scatter(pbuf, [dest], ci * _ICH + k * 16 + lanes,
                                   mask=m)
                return jnp.minimum(cnt + off[15], _CAP - 16)

            return lax.cond(jnp.any(m), compact, lambda: cnt)

        return lax.fori_loop(0, _ICH // 16, scan_step, cnt)

    cnt = lax.fori_loop(0, _B // _ICH, scan_outer, jnp.int32(0))
    nch = (cnt + 15) // 16

    def dedup_step(j, _):
        valid = (j * 16 + lanes) < cnt
        tt = jnp.where(valid, tbuf[pl.ds(j * 16, 16)], 0)
        pp = jnp.where(valid, pbuf[pl.ds(j * 16, 16)], -1)

        def cond(lost):
            return jnp.any(lost)

        def body(lost):
            plsc.store_scatter(posmap, [tt], pp, mask=lost)
            g = plsc.load_gather(posmap, [tt], mask=valid)
            return valid & (g < pp)

        lax.while_loop(cond, body, valid)
        return 0

    lax.fori_loop(0, nch, dedup_step, 0)

    def win_step(j, cnt2):
        valid = (j * 16 + lanes) < cnt
        tt = jnp.where(valid, tbuf[pl.ds(j * 16, 16)], 0)
        pp = jnp.where(valid, pbuf[pl.ds(j * 16, 16)], -1)
        g = plsc.load_gather(posmap, [tt], mask=valid)
        keep = valid & (g == pp)
        off = _psum16(jnp.where(keep, 1, 0).astype(jnp.int32))
        dest = jnp.maximum(cnt2 + off - 1, 0)
        plsc.store_scatter(tfin, [dest], tt, mask=keep)
        plsc.store_scatter(pfin, [dest], pp, mask=keep)
        return jnp.minimum(cnt2 + off[15], _CAP - 16)

    cnt2 = lax.fori_loop(0, nch, win_step, jnp.int32(0))
    nj2 = (cnt2 + 15) // 16

    zero16 = jnp.zeros((16,), jnp.int32)
    for q in range(_NBIN // 16 + 1):
        bincnt[pl.ds(q * 16, 16)] = zero16

    def bin_count(j, _):
        valid = (j * 16 + lanes) < cnt2
        tt = jnp.where(valid, tfin[pl.ds(j * 16, 16)], 0)
        bn = jnp.minimum(tt // _CCH, _NBIN - 1)
        plsc.addupdate_scatter(bincnt, [bn],
                               jnp.where(valid, 1, 0).astype(jnp.int32),
                               mask=valid)
        return 0

    lax.fori_loop(0, nj2, bin_count, 0)

    carry = jnp.int32(0)
    for q in range(_NBIN // 16 + 1):
        v = bincnt[pl.ds(q * 16, 16)]
        inc = _psum16(v)
        exc = inc - v + carry
        binoff[pl.ds(q * 16, 16)] = exc
        ctr[pl.ds(q * 16, 16)] = exc
        carry = carry + inc[15]

    def place(j, _):
        valid = (j * 16 + lanes) < cnt2
        tt = jnp.where(valid, tfin[pl.ds(j * 16, 16)], 0)
        pp = jnp.where(valid, pfin[pl.ds(j * 16, 16)], 0)
        bn = jnp.minimum(tt // _CCH, _NBIN - 1)

        def cond(active):
            return jnp.any(active)

        def body(active):
            plsc.store_scatter(claim, [bn], lanes, mask=active)
            g = plsc.load_gather(claim, [bn], mask=active)
            win = active & (g == lanes)
            slot = plsc.load_gather(ctr, [bn], mask=win)
            slot = jnp.clip(slot, 0, _CAP - 1)
            plsc.store_scatter(tbuf, [slot], tt, mask=win)
            plsc.store_scatter(pbuf, [slot], pp, mask=win)
            plsc.store_scatter(ctr, [bn], slot + 1, mask=win)
            return active & (~win)

        lax.while_loop(cond, body, valid)
        return 0

    lax.fori_loop(0, nj2, place, 0)

    def seg_of(cidx):
        f = jnp.full((16,), cidx, jnp.int32)
        so = plsc.load_gather(binoff, [f])[0]
        ns = plsc.load_gather(bincnt, [f])[0]
        return so, ns

    def prep(cidx, slot):
        so, ns = seg_of(cidx)
        ng = (ns + 15) // 16

        def gstep(v, _):
            pv = jnp.clip(pbuf[pl.ds(so + v * 16, 16)] // 2, 0, _B // 2 - 1)
            pltpu.async_copy(uv_hbm.at[pv],
                             stage.at[slot].at[pl.ds(v * 16, 16)],
                             sem_g.at[slot])
            return 0

        lax.fori_loop(0, ng, gstep, 0)
        return so, ns, ng

    def drain_gathers(ngath, slot):
        def dstep(v, _):
            pltpu.make_async_copy(uv_hbm.at[pl.ds(0, 16)],
                                  stage.at[slot].at[pl.ds(0, 16)],
                                  sem_g.at[slot]).wait()
            return 0

        lax.fori_loop(0, ngath, dstep, 0)

    def merge(lo, so, ns, slot, cslot):
        nv = (ns + 15) // 16
        cb = cbuf.at[cslot]
        st = stage.at[slot]

        def mstep(v, _):
            valid = (v * 16 + lanes) < ns
            rloc = jnp.where(valid, tbuf[pl.ds(so + v * 16, 16)] - lo, 0)
            rloc = jnp.clip(rloc, 0, _CCH - 1)
            pp = jnp.where(valid, pbuf[pl.ds(so + v * 16, 16)], 0)
            srow = v * 16 + lanes
            halfoff = (pp & 1) * _D
            for j in range(_D):
                vals = plsc.load_gather(st, [srow, halfoff + j], mask=valid)
                plsc.store_scatter(cb, [rloc, jnp.full((16,), j, jnp.int32)],
                                   vals, mask=valid)
            return 0

        lax.fori_loop(0, nv, mstep, 0)

    def in_copy(i, slot):
        return pltpu.make_async_copy(
            states_hbm.at[pl.ds(base + i * _CCH, _CCH)],
            cbuf.at[slot], sem_in.at[slot])

    def out_copy(i, slot):
        return pltpu.make_async_copy(
            cbuf.at[slot],
            out_hbm.at[pl.ds(base + i * _CCH, _CCH)], sem_out.at[slot])

    in_copy(0, 0).start()
    pr0 = prep(0, 0)

    def copy_step(i, carry):
        so_cur, ns_cur, ng_cur = carry
        slot = lax.rem(i, 2)
        nxt = lax.rem(i + 1, 2)

        @pl.when(i + 1 < _NCH)
        def _():
            @pl.when(i >= 1)
            def _():
                out_copy(i - 1, nxt).wait()
            in_copy(i + 1, nxt).start()

        nxt_carry = lax.cond(
            i + 1 < _NCH,
            lambda: prep(i + 1, nxt),
            lambda: (jnp.int32(0), jnp.int32(0), jnp.int32(0)))

        in_copy(i, slot).wait()
        drain_gathers(ng_cur, slot)
        merge(i * _CCH, so_cur, ns_cur, slot, slot)
        out_copy(i, slot).start()
        return nxt_carry

    lax.fori_loop(0, _NCH, copy_step, pr0)
    out_copy(_NCH - 2, _NCH % 2).wait()
    out_copy(_NCH - 1, 1 - _NCH % 2).wait()

    @pl.when(wid == _NW - 1)
    def _():
        so, ns, ng = prep(_NCH, 0)
        pltpu.sync_copy(states_hbm.at[pl.ds(_NW * _ROWS, _TAIL)],
                        cbuf.at[0].at[pl.ds(0, _TAIL)])
        drain_gathers(ng, 0)
        merge(_ROWS, so, ns, 0, 0)
        pltpu.sync_copy(cbuf.at[0].at[pl.ds(0, _TAIL)],
                        out_hbm.at[pl.ds(_NW * _ROWS, _TAIL)])


def kernel(states, idx, updated):
    uv = jnp.reshape(updated, (_B // 2, 2 * _D))
    mesh = plsc.VectorSubcoreMesh(core_axis_name="c", subcore_axis_name="s")
    return pl.kernel(
        _body,
        out_type=jax.ShapeDtypeStruct((_N, _D), jnp.float32),
        mesh=mesh,
        compiler_params=pltpu.CompilerParams(needs_layout_passes=False),
        scratch_types=[
            pltpu.VMEM((_ICH,), jnp.int32),
            pltpu.VMEM((_CAP + 16,), jnp.int32),
            pltpu.VMEM((_CAP + 16,), jnp.int32),
            pltpu.VMEM((_CAP + 16,), jnp.int32),
            pltpu.VMEM((_CAP + 16,), jnp.int32),
            pltpu.VMEM((_RANGE_LAST,), jnp.int32),
            pltpu.VMEM((2, _CCH, _D), jnp.float32),
            pltpu.VMEM((2, _SCAP + 16, 2 * _D), jnp.float32),
            pltpu.VMEM((128,), jnp.int32),
            pltpu.VMEM((128,), jnp.int32),
            pltpu.VMEM((128,), jnp.int32),
            pltpu.VMEM((128,), jnp.int32),
            pltpu.SemaphoreType.DMA((2,)),
            pltpu.SemaphoreType.DMA((2,)),
            pltpu.SemaphoreType.DMA((2,)),
        ],
    )(states, idx, uv)
